# baseline (speedup 1.0000x reference)
import functools
import math

import jax
import jax.numpy as jnp
from jax import lax
from jax.experimental import pallas as pl
from jax.experimental.pallas import tpu as pltpu
from jax.experimental.pallas import tpu_sc as plsc

VOCAB = 100000
D_MODEL = 128
SCALE = math.sqrt(float(D_MODEL))

NUM_CORES = 2
NUM_SUBCORES = 16
NUM_WORKERS = NUM_CORES * NUM_SUBCORES
LANES = 16

B_TOTAL = 1024 * 200
BPW = B_TOTAL // NUM_WORKERS
CHUNK = 64
NCHUNK = BPW // CHUNK
K = 10
NGROUP = NCHUNK // K

_mesh = plsc.VectorSubcoreMesh(core_axis_name="c", subcore_axis_name="s")


@functools.partial(
    pl.kernel,
    out_type=jax.ShapeDtypeStruct((B_TOTAL, D_MODEL), jnp.float32),
    mesh=_mesh,
    scratch_types=[
        pltpu.VMEM((NCHUNK, CHUNK), jnp.int32),
        pltpu.VMEM((K, CHUNK, D_MODEL), jnp.float32),
        pltpu.SemaphoreType.DMA((K,)),
        pltpu.SemaphoreType.DMA((K,)),
    ],
)
def _embed_sc(x_hbm, tbl_hbm, out_hbm, idx_v, rows_v, gsem, wsem):
    wid = lax.axis_index("s") * NUM_CORES + lax.axis_index("c")
    base = wid * BPW

    pltpu.sync_copy(x_hbm.at[wid], idx_v)

    def group_body(g, carry):
        for b in range(K):
            i = g * K + b

            @pl.when(g > 0)
            def _drain_prev():
                pltpu.make_async_copy(
                    rows_v.at[b],
                    out_hbm.at[pl.ds(base + (i - K) * CHUNK, CHUNK)],
                    wsem.at[b],
                ).wait()

            pltpu.make_async_copy(
                tbl_hbm.at[idx_v.at[i]], rows_v.at[b], gsem.at[b]
            ).start()

        for b in range(K):
            i = g * K + b
            pltpu.make_async_copy(
                tbl_hbm.at[idx_v.at[i]], rows_v.at[b], gsem.at[b]
            ).wait()

            def _scale(r, c, b=b):
                for rr in range(4):
                    for j in range(D_MODEL // LANES):
                        sl = pl.ds(j * LANES, LANES)
                        rows_v[b, 4 * r + rr, sl] = rows_v[b, 4 * r + rr, sl] * SCALE
                return c

            lax.fori_loop(0, CHUNK // 4, _scale, 0)

            pltpu.make_async_copy(
                rows_v.at[b], out_hbm.at[pl.ds(base + i * CHUNK, CHUNK)], wsem.at[b]
            ).start()
        return carry

    lax.fori_loop(0, NGROUP, group_body, 0)

    for b in range(K):
        i = (NGROUP - 1) * K + b
        pltpu.make_async_copy(
            rows_v.at[b], out_hbm.at[pl.ds(base + i * CHUNK, CHUNK)], wsem.at[b]
        ).wait()


def kernel(x, emb_weight):
    x_flat = x.reshape(NUM_WORKERS, NCHUNK, CHUNK).astype(jnp.int32)
    out = _embed_sc(x_flat, emb_weight)
    return out.reshape(x.shape + (D_MODEL,))


# speedup vs baseline: 1.0128x; 1.0128x over previous
import functools
import math

import jax
import jax.numpy as jnp
from jax import lax
from jax.experimental import pallas as pl
from jax.experimental.pallas import tpu as pltpu
from jax.experimental.pallas import tpu_sc as plsc

VOCAB = 100000
D_MODEL = 128
SCALE = math.sqrt(float(D_MODEL))

NUM_CORES = 2
NUM_SUBCORES = 16
NUM_WORKERS = NUM_CORES * NUM_SUBCORES
LANES = 16

B_TOTAL = 1024 * 200
BPW = B_TOTAL // NUM_WORKERS
CHUNK = 64
NCHUNK = BPW // CHUNK
K = 10
NGROUP = NCHUNK // K

_mesh = plsc.VectorSubcoreMesh(core_axis_name="c", subcore_axis_name="s")


@functools.partial(
    pl.kernel,
    out_type=jax.ShapeDtypeStruct((B_TOTAL, D_MODEL), jnp.float32),
    mesh=_mesh,
    scratch_types=[
        pltpu.VMEM((NCHUNK, CHUNK), jnp.int32),
        pltpu.VMEM((K, CHUNK, D_MODEL), jnp.float32),
        pltpu.SemaphoreType.DMA((K,)),
        pltpu.SemaphoreType.DMA((K,)),
    ],
)
def _embed_sc(x_hbm, tbl_hbm, out_hbm, idx_v, rows_v, gsem, wsem):
    wid = lax.axis_index("s") * NUM_CORES + lax.axis_index("c")
    base = wid * BPW

    pltpu.sync_copy(x_hbm.at[wid], idx_v)

    def group_body(g, carry):
        for b in range(K):
            i = g * K + b

            @pl.when(g > 0)
            def _drain_prev():
                pltpu.make_async_copy(
                    rows_v.at[b],
                    out_hbm.at[pl.ds(base + (i - K) * CHUNK, CHUNK)],
                    wsem.at[b],
                ).wait()

            pltpu.make_async_copy(
                tbl_hbm.at[idx_v.at[i]], rows_v.at[b], gsem.at[b]
            ).start()

        for b in range(K):
            i = g * K + b
            pltpu.make_async_copy(
                tbl_hbm.at[idx_v.at[i]], rows_v.at[b], gsem.at[b]
            ).wait()

            def _scale(r, c, b=b):
                for rr in range(2):
                    for j in range(D_MODEL // LANES):
                        sl = pl.ds(j * LANES, LANES)
                        rows_v[b, 2 * r + rr, sl] = rows_v[b, 2 * r + rr, sl] * SCALE
                return c

            lax.fori_loop(0, CHUNK // 2, _scale, 0)

            pltpu.make_async_copy(
                rows_v.at[b], out_hbm.at[pl.ds(base + i * CHUNK, CHUNK)], wsem.at[b]
            ).start()
        return carry

    lax.fori_loop(0, NGROUP, group_body, 0)

    for b in range(K):
        i = (NGROUP - 1) * K + b
        pltpu.make_async_copy(
            rows_v.at[b], out_hbm.at[pl.ds(base + i * CHUNK, CHUNK)], wsem.at[b]
        ).wait()


def kernel(x, emb_weight):
    x_flat = x.reshape(NUM_WORKERS, NCHUNK, CHUNK).astype(jnp.int32)
    out = _embed_sc(x_flat, emb_weight)
    return out.reshape(x.shape + (D_MODEL,))
